# baseline (speedup 1.0000x reference)
import functools

import jax
import jax.numpy as jnp
from jax import lax
from jax.experimental import pallas as pl
from jax.experimental.pallas import tpu as pltpu
from jax.experimental.pallas import tpu_sc as plsc

F = 16
NC = 2
NS = 16
NW = NC * NS
SUB = 128
NB = 4


def _mesh():
    return plsc.VectorSubcoreMesh(core_axis_name="c", subcore_axis_name="s",
                                  num_cores=NC, num_subcores=NS)


def _sc_degree(n_pad, nsub, k0):
    rpt = n_pad // NS
    k1 = 2 * nsub - k0

    @functools.partial(
        pl.kernel,
        out_type=jax.ShapeDtypeStruct((NC, n_pad, F), jnp.float32),
        mesh=_mesh(),
        compiler_params=pltpu.CompilerParams(use_tc_tiling_on_sc=False),
        scratch_types=[
            pltpu.VMEM((max(2 * nsub - k0, k0), SUB), jnp.int32),
            pltpu.VMEM((SUB, F), jnp.float32),
            pltpu.VMEM_SHARED((n_pad, F), jnp.float32),
            pltpu.SemaphoreType.DMA,
        ],
    )
    def deg_kernel(dst_hbm, ones_hbm, zeros_hbm, out_hbm, didx_v, ones_v, acc_sh,
                   sem):
        c = lax.axis_index("c")
        s = lax.axis_index("s")
        nsub_c = jnp.where(c == 0, k0, k1)
        base = c * NS * k0 + s * nsub_c
        kmax = max(k0, k1)
        pltpu.sync_copy(dst_hbm.at[pl.ds(base, kmax)], didx_v)
        pltpu.sync_copy(ones_hbm, ones_v)
        pltpu.sync_copy(zeros_hbm.at[pl.ds(s * rpt, rpt)],
                        acc_sh.at[pl.ds(s * rpt, rpt)])
        plsc.subcore_barrier()

        def body(j, carry):
            pltpu.async_copy(ones_v, acc_sh.at[didx_v.at[j]], sem, add=True)
            return carry

        lax.fori_loop(0, nsub_c, body, 0)

        def drain(j, carry):
            pltpu.make_async_copy(ones_v, acc_sh.at[didx_v.at[0]], sem).wait()
            return carry

        lax.fori_loop(0, nsub_c, drain, 0)
        plsc.subcore_barrier()
        pltpu.sync_copy(acc_sh.at[pl.ds(s * rpt, rpt)],
                        out_hbm.at[c, pl.ds(s * rpt, rpt)])

    return deg_kernel


def _sc_scatter(n, n_pad, nsub, k0):
    rpt = n_pad // NS
    k1 = 2 * nsub - k0
    kmax = max(k0, k1)

    @functools.partial(
        pl.kernel,
        out_type=jax.ShapeDtypeStruct((NC, n_pad, F), jnp.float32),
        mesh=_mesh(),
        compiler_params=pltpu.CompilerParams(use_tc_tiling_on_sc=False),
        scratch_types=[
            pltpu.VMEM((max(2 * nsub - k0, k0), SUB), jnp.int32),
            pltpu.VMEM((max(2 * nsub - k0, k0), SUB), jnp.int32),
            pltpu.VMEM((2 * NB, SUB, F), jnp.float32),
            pltpu.VMEM_SHARED((n_pad, F), jnp.float32),
            pltpu.SemaphoreType.DMA,
            pltpu.SemaphoreType.DMA,
        ],
    )
    def scat_kernel(src_hbm, dst_hbm, rows_hbm, zeros_hbm, out_hbm,
                    sidx_v, didx_v, rows_v, acc_sh, gsem, ssem):
        c = lax.axis_index("c")
        s = lax.axis_index("s")
        T = 2 * NB
        nsub_c = jnp.where(c == 0, k0, k1)
        base = c * NS * k0 + s * nsub_c
        pltpu.sync_copy(src_hbm.at[pl.ds(base, kmax)], sidx_v)
        pltpu.sync_copy(dst_hbm.at[pl.ds(base, kmax)], didx_v)
        pltpu.sync_copy(zeros_hbm.at[pl.ds(s * rpt, rpt)],
                        acc_sh.at[pl.ds(s * rpt, rpt)])
        plsc.subcore_barrier()

        def wait_gather(j, b):
            pltpu.make_async_copy(rows_hbm.at[sidx_v.at[j]],
                                  rows_v.at[b], gsem).wait()

        def wait_scatter(b):
            pltpu.make_async_copy(rows_v.at[b],
                                  acc_sh.at[didx_v.at[0]], ssem).wait()

        for b in range(NB):
            pltpu.async_copy(rows_hbm.at[sidx_v.at[b]], rows_v.at[b], gsem)
        for b in range(T):
            wait_gather(b, b)
            pltpu.async_copy(rows_v.at[b], acc_sh.at[didx_v.at[b]], ssem,
                             add=True)
            if b >= NB:
                wait_scatter(b - NB)
            pltpu.async_copy(rows_hbm.at[sidx_v.at[b + NB]],
                             rows_v.at[(b + NB) % T], gsem)

        def body(jo, carry):
            for b in range(T):
                j = jo * T + b
                wait_gather(j, b)
                pltpu.async_copy(rows_v.at[b], acc_sh.at[didx_v.at[j]], ssem,
                                 add=True)
                wait_scatter((b + NB) % T)
                jn = jnp.minimum(j + NB, nsub_c - 1)
                pltpu.async_copy(rows_hbm.at[sidx_v.at[jn]],
                                 rows_v.at[(b + NB) % T], gsem)
            return carry

        lax.fori_loop(1, nsub_c // T, body, 0)
        for b in range(NB):
            wait_scatter(b)
            wait_gather(0, b)
        plsc.subcore_barrier()
        pltpu.sync_copy(acc_sh.at[pl.ds(s * rpt, rpt)],
                        out_hbm.at[c, pl.ds(s * rpt, rpt)])

    return scat_kernel


def _tc0_body(x_ref, w1_ref, xw_ref):
    n = x_ref.shape[0]
    n_pad = xw_ref.shape[0]
    xw_ref[pl.ds(0, n), :] = jnp.dot(x_ref[...], w1_ref[...],
                                     preferred_element_type=jnp.float32)
    xw_ref[pl.ds(n, n_pad - n), :] = jnp.zeros((n_pad - n, xw_ref.shape[1]),
                                               jnp.float32)


def _tc1_body(p_ref, xw_ref, xw1s_ref, dis_ref):
    deg = p_ref[0] + p_ref[1] + 1.0
    dis = lax.rsqrt(deg)
    dis_ref[...] = dis
    xw1s_ref[...] = xw_ref[...] * dis


def _tc2_body(p_ref, xw1s, dis, b1, w2_ref, out_ref):
    h = jnp.maximum(dis[...] * (p_ref[0] + p_ref[1] + xw1s[...]) + b1[...],
                    0.0)
    hw = jnp.dot(h, w2_ref[...], preferred_element_type=jnp.float32)
    out_ref[...] = hw * dis[...]


def _tc3_body(p_ref, hw2s, dis, b2, out_ref):
    n = out_ref.shape[0]
    logits = (dis[pl.ds(0, n), :]
              * (p_ref[0, pl.ds(0, n), :] + p_ref[1, pl.ds(0, n), :]
                 + hw2s[pl.ds(0, n), :])
              + b2[...])
    m = jnp.max(logits, axis=1, keepdims=True)
    lse = jnp.log(jnp.sum(jnp.exp(logits - m), axis=1, keepdims=True)) + m
    out_ref[...] = logits - lse


def kernel(x, edge_index, W1, b1, W2, b2):
    n, d = x.shape
    f = W1.shape[1]
    e = edge_index.shape[1]
    assert f == F and W2.shape[1] == F

    e_pad = -(-e // (NW * SUB * 8)) * (NW * SUB * 8)
    nsub = e_pad // (NW * SUB)
    n_pad = -(-(n + 1) // (NS * 8)) * (NS * 8)
    src = edge_index[0]
    dst = edge_index[1]
    if e_pad > e:
        src = jnp.concatenate([src, jnp.zeros((e_pad - e,), src.dtype)])
        dst = jnp.concatenate([dst, jnp.full((e_pad - e,), n, dst.dtype)])
    src2d = src.reshape(NW * nsub, SUB).astype(jnp.int32)
    dst2d = dst.reshape(NW * nsub, SUB).astype(jnp.int32)
    zeros = jnp.zeros((n_pad, F), jnp.float32)
    ones = jnp.ones((SUB, F), jnp.float32)

    tc0 = pl.pallas_call(
        _tc0_body,
        out_shape=jax.ShapeDtypeStruct((n_pad, F), jnp.float32),
    )
    xw1 = tc0(x, W1)

    k0 = max(8, int(round(0.55 * 2 * nsub / 8)) * 8)
    degp = _sc_degree(n_pad, nsub, k0)(dst2d, ones, zeros)

    tc1 = pl.pallas_call(
        _tc1_body,
        out_shape=[jax.ShapeDtypeStruct((n_pad, F), jnp.float32),
                   jax.ShapeDtypeStruct((n_pad, F), jnp.float32)],
    )
    xw1s, dis = tc1(degp, xw1)

    scat = _sc_scatter(n, n_pad, nsub, k0)
    s1 = scat(src2d, dst2d, xw1s, zeros)

    tc2 = pl.pallas_call(
        _tc2_body,
        out_shape=jax.ShapeDtypeStruct((n_pad, F), jnp.float32),
    )
    hw2s = tc2(s1, xw1s, dis, b1.reshape(1, F), W2)

    s2 = scat(src2d, dst2d, hw2s, zeros)

    tc3 = pl.pallas_call(
        _tc3_body,
        out_shape=jax.ShapeDtypeStruct((n, F), jnp.float32),
    )
    return tc3(s2, hw2s, dis, b2.reshape(1, F))
